# baseline (speedup 1.0000x reference)
import jax
import jax.numpy as jnp
from jax import lax
from jax.experimental import pallas as pl
from jax.experimental.pallas import tpu as pltpu

_CHUNK = 4096
_NQ = 8


def _mdma_kernel(w_ref, o_ref, buf_ref, sems):
    L, C = w_ref.shape
    B = o_ref.shape[1]
    n_chunks = B // _CHUNK

    def start_slab(l):
        row = w_ref[pl.ds(l, 1), :]
        buf_ref[l % 2] = jnp.broadcast_to(row[:, None, :], (1, _CHUNK, C))
        for k in range(n_chunks):
            pltpu.make_async_copy(
                buf_ref.at[l % 2],
                o_ref.at[pl.ds(l, 1), pl.ds(k * _CHUNK, _CHUNK)],
                sems.at[(l * n_chunks + k) % _NQ],
            ).start()

    def wait_slab(l):
        for k in range(n_chunks):
            pltpu.make_async_copy(
                buf_ref.at[l % 2],
                o_ref.at[pl.ds(l, 1), pl.ds(k * _CHUNK, _CHUNK)],
                sems.at[(l * n_chunks + k) % _NQ],
            ).wait()

    start_slab(0)
    start_slab(1)

    def body(i, carry):
        wait_slab(i - 2)
        start_slab(i)
        return carry

    lax.fori_loop(2, L, body, 0)
    wait_slab(L - 2)
    wait_slab(L - 1)


def kernel(x, pos_embed_weight):
    B = x.shape[0]
    L, C = pos_embed_weight.shape
    lbc = pl.pallas_call(
        _mdma_kernel,
        in_specs=[pl.BlockSpec(memory_space=pltpu.MemorySpace.VMEM)],
        out_specs=pl.BlockSpec(memory_space=pl.ANY),
        out_shape=jax.ShapeDtypeStruct((L, B, C), jnp.float32),
        scratch_shapes=[
            pltpu.VMEM((2, 1, _CHUNK, C), jnp.float32),
            pltpu.SemaphoreType.DMA((_NQ,)),
        ],
    )(pos_embed_weight)
    return jnp.transpose(lbc, (1, 2, 0))


# speedup vs baseline: 1.0080x; 1.0080x over previous
import jax
import jax.numpy as jnp
from jax.experimental import pallas as pl

_B_BLOCK = 4096


def _bcast_kernel(w_ref, o_ref):
    l = pl.program_id(0)
    row = w_ref[pl.ds(l, 1), :]
    o_ref[...] = jnp.broadcast_to(row[:, None, :], o_ref.shape)


def kernel(x, pos_embed_weight):
    B = x.shape[0]
    L, C = pos_embed_weight.shape
    lbc = pl.pallas_call(
        _bcast_kernel,
        grid=(L, B // _B_BLOCK),
        in_specs=[pl.BlockSpec((L, C), lambda l, i: (0, 0))],
        out_specs=pl.BlockSpec((1, _B_BLOCK, C), lambda l, i: (l, i, 0)),
        out_shape=jax.ShapeDtypeStruct((L, B, C), jnp.float32),
    )(pos_embed_weight)
    return jnp.transpose(lbc, (1, 2, 0))
